# baseline (speedup 1.0000x reference)
import functools

import jax
import jax.numpy as jnp
from jax.experimental import pallas as pl
from jax.experimental.pallas import tpu as pltpu
from jax.experimental.pallas import tpu_sc as plsc

_GATHER_WIN = 128
_TABLE_BLK = 2000


def _eye(k):
    r = jax.lax.broadcasted_iota(jnp.int32, (k, k), 0)
    c = jax.lax.broadcasted_iota(jnp.int32, (k, k), 1)
    return (r == c).astype(jnp.float32)


def _sc_gather(table128, ids2d):
    n = ids2d.shape[1]
    dw = table128.shape[1]
    mesh = plsc.VectorSubcoreMesh(core_axis_name="core", subcore_axis_name="subcore")

    @functools.partial(
        pl.kernel,
        out_type=jax.ShapeDtypeStruct((n, dw), table128.dtype),
        mesh=mesh,
    )
    def gather_kernel(emb_hbm, ids_hbm, out_hbm):
        def body(i_vmem, o_vmem):
            pltpu.sync_copy(emb_hbm.at[i_vmem.at[0]], o_vmem)

        pltpu.emit_pipeline(
            body,
            grid=(n // _GATHER_WIN,),
            in_specs=[pl.BlockSpec((1, _GATHER_WIN), lambda i: (0, i))],
            out_specs=[pl.BlockSpec((_GATHER_WIN, dw), lambda i: (i, 0))],
            core_axis_name=("core", "subcore"),
            dimension_semantics=(pltpu.PARALLEL,),
        )(ids_hbm, out_hbm)

    return gather_kernel(table128, ids2d)


def _tc_combine(tokens_t, pe3, topo_t, W):
    s, d, b = tokens_t.shape
    f = topo_t.shape[1]
    dw = pe3.shape[2]

    def body(tok_ref, pe_ref, topo_ref, w_ref, out_ref):
        pe_t = jax.lax.dot_general(
            _eye(d), pe_ref[0, :, :d], (((1,), (1,)), ((), ())),
            preferred_element_type=jnp.float32,
            precision=jax.lax.Precision.DEFAULT,
        )
        proj = jax.lax.dot_general(
            w_ref[...], topo_ref[0], (((1,), (0,)), ((), ())),
            preferred_element_type=jnp.float32,
        )
        out_ref[0] = tok_ref[0] + pe_t + proj

    return pl.pallas_call(
        body,
        grid=(s,),
        in_specs=[
            pl.BlockSpec((1, d, b), lambda i: (i, 0, 0)),
            pl.BlockSpec((1, b, dw), lambda i: (i, 0, 0)),
            pl.BlockSpec((1, f, b), lambda i: (i, 0, 0)),
            pl.BlockSpec((d, f), lambda i: (0, 0)),
        ],
        out_specs=pl.BlockSpec((1, d, b), lambda i: (i, 0, 0)),
        out_shape=jax.ShapeDtypeStruct((s, d, b), jnp.float32),
    )(tokens_t, pe3, topo_t, W)


def kernel(tokens, ids, topo_feats, id_emb, W_proj):
    b, s, d = tokens.shape
    n = b * s
    tokens_t = jnp.transpose(tokens, (1, 2, 0))
    topo_t = jnp.transpose(topo_feats, (1, 2, 0))
    ids_sm = ids.T.reshape(1, n).astype(jnp.int32)
    table128 = jnp.transpose(jnp.pad(id_emb.T, ((0, 128 - d), (0, 0))))
    pe3 = _sc_gather(table128, ids_sm).reshape(s, b, 128)
    out_t = _tc_combine(tokens_t, pe3, topo_t, W_proj)
    return jnp.transpose(out_t, (2, 0, 1))


# speedup vs baseline: 1.3340x; 1.3340x over previous
import functools

import jax
import jax.numpy as jnp
from jax.experimental import pallas as pl
from jax.experimental.pallas import tpu as pltpu
from jax.experimental.pallas import tpu_sc as plsc

_GATHER_WIN = 128
_TABLE_BLK = 2000


def _eye(k):
    r = jax.lax.broadcasted_iota(jnp.int32, (k, k), 0)
    c = jax.lax.broadcasted_iota(jnp.int32, (k, k), 1)
    return (r == c).astype(jnp.float32)


def _sc_gather(table128, ids2d):
    n = ids2d.shape[1]
    dw = table128.shape[1]
    mesh = plsc.VectorSubcoreMesh(core_axis_name="core", subcore_axis_name="subcore")

    @functools.partial(
        pl.kernel,
        out_type=jax.ShapeDtypeStruct((n, dw), table128.dtype),
        mesh=mesh,
    )
    def gather_kernel(emb_hbm, ids_hbm, out_hbm):
        def body(i_vmem, o_vmem):
            pltpu.sync_copy(emb_hbm.at[i_vmem.at[0]], o_vmem)

        pltpu.emit_pipeline(
            body,
            grid=(n // _GATHER_WIN,),
            in_specs=[pl.BlockSpec((1, _GATHER_WIN), lambda i: (0, i))],
            out_specs=[pl.BlockSpec((_GATHER_WIN, dw), lambda i: (i, 0))],
            core_axis_name=("core", "subcore"),
            dimension_semantics=(pltpu.PARALLEL,),
        )(ids_hbm, out_hbm)

    return gather_kernel(table128, ids2d)


_SB = 4


def _tc_combine(tokens_t, pe3, topo_t, W):
    s, d, b = tokens_t.shape
    f = topo_t.shape[1]
    dw = pe3.shape[2]

    def body(tok_ref, pe_ref, topo_ref, w_ref, out_ref):
        for j in range(_SB):
            pe_t = jax.lax.dot_general(
                _eye(d), pe_ref[j, :, :d], (((1,), (1,)), ((), ())),
                preferred_element_type=jnp.float32,
                precision=jax.lax.Precision.DEFAULT,
            )
            proj = jax.lax.dot_general(
                w_ref[...], topo_ref[j], (((1,), (0,)), ((), ())),
                preferred_element_type=jnp.float32,
            )
            out_ref[j] = tok_ref[j] + pe_t + proj

    return pl.pallas_call(
        body,
        grid=(s // _SB,),
        in_specs=[
            pl.BlockSpec((_SB, d, b), lambda i: (i, 0, 0)),
            pl.BlockSpec((_SB, b, dw), lambda i: (i, 0, 0)),
            pl.BlockSpec((_SB, f, b), lambda i: (i, 0, 0)),
            pl.BlockSpec((d, f), lambda i: (0, 0)),
        ],
        out_specs=pl.BlockSpec((_SB, d, b), lambda i: (i, 0, 0)),
        out_shape=jax.ShapeDtypeStruct((s, d, b), jnp.float32),
        compiler_params=pltpu.CompilerParams(
            dimension_semantics=("parallel",),
        ),
    )(tokens_t, pe3, topo_t, W)


def kernel(tokens, ids, topo_feats, id_emb, W_proj):
    b, s, d = tokens.shape
    n = b * s
    tokens_t = jnp.transpose(tokens, (1, 2, 0))
    topo_t = jnp.transpose(topo_feats, (1, 2, 0))
    ids_sm = ids.T.reshape(1, n).astype(jnp.int32)
    table128 = jnp.transpose(jnp.pad(id_emb.T, ((0, 128 - d), (0, 0))))
    pe3 = _sc_gather(table128, ids_sm).reshape(s, b, 128)
    out_t = _tc_combine(tokens_t, pe3, topo_t, W_proj)
    return jnp.transpose(out_t, (2, 0, 1))
